# baseline (speedup 1.0000x reference)
import functools

import jax
import jax.numpy as jnp
from jax import lax
from jax.experimental import pallas as pl
from jax.experimental.pallas import tpu as pltpu
from jax.experimental.pallas import tpu_sc as plsc

NC = 2
NS = 16
NW = NC * NS
CH = 128
CW = 16
ZB = 128


def _mesh():
    return plsc.VectorSubcoreMesh(
        core_axis_name="c", subcore_axis_name="s",
        num_cores=NC, num_subcores=NS)


@functools.lru_cache(maxsize=None)
def _make_sc_agg(n_nodes, d, t0, t1, acc_rows, zr):
    hd = d // 2
    tmax = max(t0, t1)
    ept = tmax * CH
    out_type = jax.ShapeDtypeStruct((2, NC, acc_rows, hd), jnp.float32)
    scratch = (
        pltpu.VMEM((ept,), jnp.int32),
        pltpu.VMEM((ept,), jnp.int32),
        pltpu.VMEM((CH, hd), jnp.float32),
        pltpu.VMEM((ZB, hd), jnp.float32),
        pltpu.VMEM_SHARED((acc_rows, hd), jnp.float32),
        pltpu.SemaphoreType.DMA,
    )

    def body(h0_hbm, h1_hbm, src_hbm, dst_hbm, S_hbm, src_v, dst_v, buf,
             zbuf, acc, sem):
        cid = lax.axis_index("c")
        sid = lax.axis_index("s")
        r0 = sid * zr
        tc = jnp.where(cid == 0, t0, t1)
        off = cid * (NS * t0 * CH) + sid * tc * CH

        def zstep(i, c):
            zbuf[i // (hd // 16), pl.ds((i % (hd // 16)) * 16, 16)] = (
                jnp.zeros((16,), jnp.float32))
            return c
        lax.fori_loop(0, ZB * (hd // 16), zstep, 0)

        pltpu.sync_copy(src_hbm.at[pl.ds(off, ept)], src_v)
        pltpu.sync_copy(dst_hbm.at[pl.ds(off, ept)], dst_v)

        for hf, h_hbm in enumerate((h0_hbm, h1_hbm)):
            for k in range(zr // ZB):
                pltpu.sync_copy(zbuf, acc.at[pl.ds(r0 + k * ZB, ZB)])
            plsc.subcore_barrier()

            def step(j, c):
                pltpu.async_copy(
                    h_hbm.at[src_v.at[pl.ds(j * CH, CH)]], buf, sem).wait()
                pltpu.sync_copy(
                    buf, acc.at[dst_v.at[pl.ds(j * CH, CH)]], add=True)
                return c
            lax.fori_loop(0, tc, step, 0)
            plsc.subcore_barrier()

            pltpu.sync_copy(acc.at[pl.ds(r0, zr)],
                            S_hbm.at[hf, cid, pl.ds(r0, zr)])

    return pl.kernel(
        body, out_type, mesh=_mesh(), scratch_types=scratch,
        compiler_params=pltpu.CompilerParams(use_tc_tiling_on_sc=False))


@functools.lru_cache(maxsize=None)
def _make_sc_cnt(t0, t1, acc_rows, zr):
    tmax = max(t0, t1)
    ept = tmax * CH
    out_type = jax.ShapeDtypeStruct((NC, acc_rows, CW), jnp.float32)
    scratch = (
        pltpu.VMEM((ept,), jnp.int32),
        pltpu.VMEM((CH, CW), jnp.float32),
        pltpu.VMEM((ZB, CW), jnp.float32),
        pltpu.VMEM_SHARED((acc_rows, CW), jnp.float32),
        pltpu.SemaphoreType.DMA,
    )

    def body(dst_hbm, C_hbm, dst_v, ones_v, z16, cacc, sem):
        cid = lax.axis_index("c")
        sid = lax.axis_index("s")
        r0 = sid * zr
        tc = jnp.where(cid == 0, t0, t1)
        off = cid * (NS * t0 * CH) + sid * tc * CH

        def ostep(i, c):
            ones_v[i] = jnp.ones((16,), jnp.float32)
            return c
        lax.fori_loop(0, CH, ostep, 0)

        def zstep(i, c):
            z16[i] = jnp.zeros((16,), jnp.float32)
            return c
        lax.fori_loop(0, ZB, zstep, 0)

        for k in range(zr // ZB):
            pltpu.sync_copy(z16, cacc.at[pl.ds(r0 + k * ZB, ZB)])
        plsc.subcore_barrier()

        pltpu.sync_copy(dst_hbm.at[pl.ds(off, ept)], dst_v)

        def step(j, c):
            pltpu.sync_copy(
                ones_v, cacc.at[dst_v.at[pl.ds(j * CH, CH)]], add=True)
            return c
        lax.fori_loop(0, tc, step, 0)
        plsc.subcore_barrier()

        pltpu.sync_copy(cacc.at[pl.ds(r0, zr)], C_hbm.at[cid, pl.ds(r0, zr)])

    return pl.kernel(
        body, out_type, mesh=_mesh(), scratch_types=scratch,
        compiler_params=pltpu.CompilerParams(use_tc_tiling_on_sc=False))


@functools.lru_cache(maxsize=None)
def _make_tc_combine(n_nodes, d, blk):
    grid = n_nodes // blk
    hd = d // 2

    def body(s00, s01, s10, s11, c0, c1, h_ref, wl_ref, bl_ref, wr_ref,
             a_ref, o_ref):
        sa = s00[0, 0] + s01[0, 0]
        sb = s10[0, 0] + s11[0, 0]
        deg = c0[0][:, :1] + c1[0][:, :1]
        inv = 1.0 / jnp.maximum(deg, 1.0)
        agg = jnp.concatenate([sa * inv, sb * inv], axis=1)
        out = (jnp.dot(agg, wl_ref[...], preferred_element_type=jnp.float32)
               + jnp.dot(h_ref[...], wr_ref[...],
                         preferred_element_type=jnp.float32)
               + bl_ref[...])
        o_ref[...] = jnp.where(out > 0, out, a_ref[...] * out)

    return pl.pallas_call(
        body,
        grid=(grid,),
        in_specs=[
            pl.BlockSpec((1, 1, blk, hd), lambda i: (0, 0, i, 0)),
            pl.BlockSpec((1, 1, blk, hd), lambda i: (0, 1, i, 0)),
            pl.BlockSpec((1, 1, blk, hd), lambda i: (1, 0, i, 0)),
            pl.BlockSpec((1, 1, blk, hd), lambda i: (1, 1, i, 0)),
            pl.BlockSpec((1, blk, CW), lambda i: (0, i, 0)),
            pl.BlockSpec((1, blk, CW), lambda i: (1, i, 0)),
            pl.BlockSpec((blk, d), lambda i: (i, 0)),
            pl.BlockSpec((d, d), lambda i: (0, 0)),
            pl.BlockSpec((1, d), lambda i: (0, 0)),
            pl.BlockSpec((d, d), lambda i: (0, 0)),
            pl.BlockSpec((1, d), lambda i: (0, 0)),
        ],
        out_specs=pl.BlockSpec((blk, d), lambda i: (i, 0)),
        out_shape=jax.ShapeDtypeStruct((n_nodes, d), jnp.float32),
    )


def kernel(x, edge_index, Wl0, bl0, Wr0, a0, Wl1, bl1, Wr1, a1,
           Wl2, bl2, Wr2, a2):
    n, d = x.shape
    e = edge_index.shape[1]

    tt = max(4, -(-e // (NS * CH)))
    t0 = max(1, round(tt * 0.60))
    t1 = tt - t0
    epad = NS * tt * CH + max(t0, t1) * CH
    zr = ZB * (-(-(n + 1) // (NS * ZB)))
    acc_rows = NS * zr

    ei = edge_index.astype(jnp.int32)
    pad = epad - e
    src = jnp.concatenate([ei[0], jnp.zeros((pad,), jnp.int32)])
    dst = jnp.concatenate([ei[1], jnp.full((pad,), n, jnp.int32)])

    sc_cnt = _make_sc_cnt(t0, t1, acc_rows, zr)
    sc_agg = _make_sc_agg(n, d, t0, t1, acc_rows, zr)
    tc = _make_tc_combine(n, d, 1000)

    hd = d // 2
    C = sc_cnt(dst)
    S = sc_agg(x[:, :hd], x[:, hd:], src, dst)
    h = tc(S, S, S, S, C, C, x,
           Wl0.T, bl0.reshape(1, d), Wr0.T, a0.reshape(1, d))
    S = sc_agg(h[:, :hd], h[:, hd:], src, dst)
    h = tc(S, S, S, S, C, C, h,
           Wl1.T, bl1.reshape(1, d), Wr1.T, a1.reshape(1, d))
    S = sc_agg(h[:, :hd], h[:, hd:], src, dst)
    h = tc(S, S, S, S, C, C, h,
           Wl2.T, bl2.reshape(1, d), Wr2.T, a2.reshape(1, d))
    return h


# speedup vs baseline: 1.0205x; 1.0205x over previous
import functools

import jax
import jax.numpy as jnp
from jax import lax
from jax.experimental import pallas as pl
from jax.experimental.pallas import tpu as pltpu
from jax.experimental.pallas import tpu_sc as plsc

NC = 2
NS = 16
NW = NC * NS
CH = 128
CW = 16
ZB = 128


def _mesh():
    return plsc.VectorSubcoreMesh(
        core_axis_name="c", subcore_axis_name="s",
        num_cores=NC, num_subcores=NS)


@functools.lru_cache(maxsize=None)
def _make_sc_agg(n_nodes, d, t0, t1, acc_rows, zr):
    hd = d // 2
    tmax = max(t0, t1)
    ept = tmax * CH
    out_type = jax.ShapeDtypeStruct((2, NC, acc_rows, hd), jnp.float32)
    scratch = (
        pltpu.VMEM((ept,), jnp.int32),
        pltpu.VMEM((ept,), jnp.int32),
        pltpu.VMEM((CH, hd), jnp.float32),
        pltpu.VMEM((ZB, hd), jnp.float32),
        pltpu.VMEM_SHARED((acc_rows, hd), jnp.float32),
        pltpu.SemaphoreType.DMA,
    )

    def body(h0_hbm, h1_hbm, src_hbm, dst_hbm, S_hbm, src_v, dst_v, buf,
             zbuf, acc, sem):
        cid = lax.axis_index("c")
        sid = lax.axis_index("s")
        r0 = sid * zr
        tc = jnp.where(cid == 0, t0, t1)
        off = cid * (NS * t0 * CH) + sid * tc * CH

        def zstep(i, c):
            zbuf[i // (hd // 16), pl.ds((i % (hd // 16)) * 16, 16)] = (
                jnp.zeros((16,), jnp.float32))
            return c
        lax.fori_loop(0, ZB * (hd // 16), zstep, 0)

        pltpu.sync_copy(src_hbm.at[pl.ds(off, ept)], src_v)
        pltpu.sync_copy(dst_hbm.at[pl.ds(off, ept)], dst_v)

        for hf, h_hbm in enumerate((h0_hbm, h1_hbm)):
            for k in range(zr // ZB):
                pltpu.sync_copy(zbuf, acc.at[pl.ds(r0 + k * ZB, ZB)])
            plsc.subcore_barrier()

            def step(j, c):
                pltpu.async_copy(
                    h_hbm.at[src_v.at[pl.ds(j * CH, CH)]], buf, sem).wait()
                pltpu.sync_copy(
                    buf, acc.at[dst_v.at[pl.ds(j * CH, CH)]], add=True)
                return c
            lax.fori_loop(0, tc, step, 0)
            plsc.subcore_barrier()

            pltpu.sync_copy(acc.at[pl.ds(r0, zr)],
                            S_hbm.at[hf, cid, pl.ds(r0, zr)])

    return pl.kernel(
        body, out_type, mesh=_mesh(), scratch_types=scratch,
        compiler_params=pltpu.CompilerParams(use_tc_tiling_on_sc=False))


@functools.lru_cache(maxsize=None)
def _make_sc_cnt(t0, t1, acc_rows, zr):
    tmax = max(t0, t1)
    ept = tmax * CH
    out_type = jax.ShapeDtypeStruct((NC, acc_rows, CW), jnp.float32)
    scratch = (
        pltpu.VMEM((ept,), jnp.int32),
        pltpu.VMEM((CH, CW), jnp.float32),
        pltpu.VMEM((ZB, CW), jnp.float32),
        pltpu.VMEM_SHARED((acc_rows, CW), jnp.float32),
        pltpu.SemaphoreType.DMA,
    )

    def body(dst_hbm, C_hbm, dst_v, ones_v, z16, cacc, sem):
        cid = lax.axis_index("c")
        sid = lax.axis_index("s")
        r0 = sid * zr
        tc = jnp.where(cid == 0, t0, t1)
        off = cid * (NS * t0 * CH) + sid * tc * CH

        def ostep(i, c):
            ones_v[i] = jnp.ones((16,), jnp.float32)
            return c
        lax.fori_loop(0, CH, ostep, 0)

        def zstep(i, c):
            z16[i] = jnp.zeros((16,), jnp.float32)
            return c
        lax.fori_loop(0, ZB, zstep, 0)

        for k in range(zr // ZB):
            pltpu.sync_copy(z16, cacc.at[pl.ds(r0 + k * ZB, ZB)])
        plsc.subcore_barrier()

        pltpu.sync_copy(dst_hbm.at[pl.ds(off, ept)], dst_v)

        def step(j, c):
            pltpu.sync_copy(
                ones_v, cacc.at[dst_v.at[pl.ds(j * CH, CH)]], add=True)
            return c
        lax.fori_loop(0, tc, step, 0)
        plsc.subcore_barrier()

        pltpu.sync_copy(cacc.at[pl.ds(r0, zr)], C_hbm.at[cid, pl.ds(r0, zr)])

    return pl.kernel(
        body, out_type, mesh=_mesh(), scratch_types=scratch,
        compiler_params=pltpu.CompilerParams(use_tc_tiling_on_sc=False))


@functools.lru_cache(maxsize=None)
def _make_tc_combine(n_nodes, d, blk):
    grid = n_nodes // blk
    hd = d // 2

    def body(s00, s01, s10, s11, c0, c1, h_ref, wl_ref, bl_ref, wr_ref,
             a_ref, o_ref):
        sa = s00[0, 0] + s01[0, 0]
        sb = s10[0, 0] + s11[0, 0]
        deg = c0[0][:, :1] + c1[0][:, :1]
        inv = 1.0 / jnp.maximum(deg, 1.0)
        agg = jnp.concatenate([sa * inv, sb * inv], axis=1)
        out = (jnp.dot(agg, wl_ref[...], preferred_element_type=jnp.float32)
               + jnp.dot(h_ref[...], wr_ref[...],
                         preferred_element_type=jnp.float32)
               + bl_ref[...])
        o_ref[...] = jnp.where(out > 0, out, a_ref[...] * out)

    return pl.pallas_call(
        body,
        grid=(grid,),
        in_specs=[
            pl.BlockSpec((1, 1, blk, hd), lambda i: (0, 0, i, 0)),
            pl.BlockSpec((1, 1, blk, hd), lambda i: (0, 1, i, 0)),
            pl.BlockSpec((1, 1, blk, hd), lambda i: (1, 0, i, 0)),
            pl.BlockSpec((1, 1, blk, hd), lambda i: (1, 1, i, 0)),
            pl.BlockSpec((1, blk, CW), lambda i: (0, i, 0)),
            pl.BlockSpec((1, blk, CW), lambda i: (1, i, 0)),
            pl.BlockSpec((blk, d), lambda i: (i, 0)),
            pl.BlockSpec((d, d), lambda i: (0, 0)),
            pl.BlockSpec((1, d), lambda i: (0, 0)),
            pl.BlockSpec((d, d), lambda i: (0, 0)),
            pl.BlockSpec((1, d), lambda i: (0, 0)),
        ],
        out_specs=pl.BlockSpec((blk, d), lambda i: (i, 0)),
        out_shape=jax.ShapeDtypeStruct((n_nodes, d), jnp.float32),
    )


def kernel(x, edge_index, Wl0, bl0, Wr0, a0, Wl1, bl1, Wr1, a1,
           Wl2, bl2, Wr2, a2):
    n, d = x.shape
    e = edge_index.shape[1]

    tt = max(4, -(-e // (NS * CH)))
    t0 = max(1, round(tt * 0.55))
    t1 = tt - t0
    epad = NS * tt * CH + max(t0, t1) * CH
    zr = ZB * (-(-(n + 1) // (NS * ZB)))
    acc_rows = NS * zr

    ei = edge_index.astype(jnp.int32)
    pad = epad - e
    src = jnp.concatenate([ei[0], jnp.zeros((pad,), jnp.int32)])
    dst = jnp.concatenate([ei[1], jnp.full((pad,), n, jnp.int32)])

    sc_cnt = _make_sc_cnt(t0, t1, acc_rows, zr)
    sc_agg = _make_sc_agg(n, d, t0, t1, acc_rows, zr)
    tc = _make_tc_combine(n, d, 1000)

    hd = d // 2
    C = sc_cnt(dst)
    S = sc_agg(x[:, :hd], x[:, hd:], src, dst)
    h = tc(S, S, S, S, C, C, x,
           Wl0.T, bl0.reshape(1, d), Wr0.T, a0.reshape(1, d))
    S = sc_agg(h[:, :hd], h[:, hd:], src, dst)
    h = tc(S, S, S, S, C, C, h,
           Wl1.T, bl1.reshape(1, d), Wr1.T, a1.reshape(1, d))
    S = sc_agg(h[:, :hd], h[:, hd:], src, dst)
    h = tc(S, S, S, S, C, C, h,
           Wl2.T, bl2.reshape(1, d), Wr2.T, a2.reshape(1, d))
    return h


# speedup vs baseline: 1.0342x; 1.0135x over previous
import functools

import jax
import jax.numpy as jnp
from jax import lax
from jax.experimental import pallas as pl
from jax.experimental.pallas import tpu as pltpu
from jax.experimental.pallas import tpu_sc as plsc

NC = 2
NS = 16
NW = NC * NS
CH = 128
CW = 16
ZB = 128


def _mesh():
    return plsc.VectorSubcoreMesh(
        core_axis_name="c", subcore_axis_name="s",
        num_cores=NC, num_subcores=NS)


@functools.lru_cache(maxsize=None)
def _make_sc_agg(n_nodes, d, t0, t1, acc_rows, zr):
    hd = d // 2
    tmax = max(t0, t1)
    ept = tmax * CH
    out_type = jax.ShapeDtypeStruct((2, NC, acc_rows, hd), jnp.float32)
    scratch = (
        pltpu.VMEM((ept,), jnp.int32),
        pltpu.VMEM((ept,), jnp.int32),
        pltpu.VMEM((CH, hd), jnp.float32),
        pltpu.VMEM((ZB, hd), jnp.float32),
        pltpu.VMEM_SHARED((acc_rows, hd), jnp.float32),
        pltpu.SemaphoreType.DMA,
    )

    def body(h0_hbm, h1_hbm, src_hbm, dst_hbm, S_hbm, src_v, dst_v, buf,
             zbuf, acc, sem):
        cid = lax.axis_index("c")
        sid = lax.axis_index("s")
        r0 = sid * zr
        tc = jnp.where(cid == 0, t0, t1)
        off = cid * (NS * t0 * CH) + sid * tc * CH

        def zstep(i, c):
            zbuf[i // (hd // 16), pl.ds((i % (hd // 16)) * 16, 16)] = (
                jnp.zeros((16,), jnp.float32))
            return c
        lax.fori_loop(0, ZB * (hd // 16), zstep, 0)

        pltpu.sync_copy(src_hbm.at[pl.ds(off, ept)], src_v)
        pltpu.sync_copy(dst_hbm.at[pl.ds(off, ept)], dst_v)

        for hf, h_hbm in enumerate((h0_hbm, h1_hbm)):
            for k in range(zr // ZB):
                pltpu.sync_copy(zbuf, acc.at[pl.ds(r0 + k * ZB, ZB)])
            plsc.subcore_barrier()

            def step(j, c):
                pltpu.async_copy(
                    h_hbm.at[src_v.at[pl.ds(j * CH, CH)]], buf, sem).wait()
                pltpu.sync_copy(
                    buf, acc.at[dst_v.at[pl.ds(j * CH, CH)]], add=True)
                return c
            lax.fori_loop(0, tc, step, 0)
            plsc.subcore_barrier()

            pltpu.sync_copy(acc.at[pl.ds(r0, zr)],
                            S_hbm.at[hf, cid, pl.ds(r0, zr)])

    return pl.kernel(
        body, out_type, mesh=_mesh(), scratch_types=scratch,
        compiler_params=pltpu.CompilerParams(use_tc_tiling_on_sc=False))


@functools.lru_cache(maxsize=None)
def _make_sc_cnt(t0, t1, acc_rows, zr):
    tmax = max(t0, t1)
    ept = tmax * CH
    out_type = jax.ShapeDtypeStruct((NC, acc_rows, CW), jnp.float32)
    scratch = (
        pltpu.VMEM((ept,), jnp.int32),
        pltpu.VMEM((CH, CW), jnp.float32),
        pltpu.VMEM((ZB, CW), jnp.float32),
        pltpu.VMEM_SHARED((acc_rows, CW), jnp.float32),
        pltpu.SemaphoreType.DMA,
    )

    def body(dst_hbm, C_hbm, dst_v, ones_v, z16, cacc, sem):
        cid = lax.axis_index("c")
        sid = lax.axis_index("s")
        r0 = sid * zr
        tc = jnp.where(cid == 0, t0, t1)
        off = cid * (NS * t0 * CH) + sid * tc * CH

        def ostep(i, c):
            ones_v[i] = jnp.ones((16,), jnp.float32)
            return c
        lax.fori_loop(0, CH, ostep, 0)

        def zstep(i, c):
            z16[i] = jnp.zeros((16,), jnp.float32)
            return c
        lax.fori_loop(0, ZB, zstep, 0)

        for k in range(zr // ZB):
            pltpu.sync_copy(z16, cacc.at[pl.ds(r0 + k * ZB, ZB)])
        plsc.subcore_barrier()

        pltpu.sync_copy(dst_hbm.at[pl.ds(off, ept)], dst_v)

        def step(j, c):
            pltpu.sync_copy(
                ones_v, cacc.at[dst_v.at[pl.ds(j * CH, CH)]], add=True)
            return c
        lax.fori_loop(0, tc, step, 0)
        plsc.subcore_barrier()

        pltpu.sync_copy(cacc.at[pl.ds(r0, zr)], C_hbm.at[cid, pl.ds(r0, zr)])

    return pl.kernel(
        body, out_type, mesh=_mesh(), scratch_types=scratch,
        compiler_params=pltpu.CompilerParams(use_tc_tiling_on_sc=False))


@functools.lru_cache(maxsize=None)
def _make_tc_combine(n_nodes, d, blk):
    grid = n_nodes // blk
    hd = d // 2

    def body(s00, s01, s10, s11, c0, c1, h_ref, wl_ref, bl_ref, wr_ref,
             a_ref, o_ref):
        sa = s00[0, 0] + s01[0, 0]
        sb = s10[0, 0] + s11[0, 0]
        deg = c0[0][:, :1] + c1[0][:, :1]
        inv = 1.0 / jnp.maximum(deg, 1.0)
        agg = jnp.concatenate([sa * inv, sb * inv], axis=1)
        out = (jnp.dot(agg, wl_ref[...], preferred_element_type=jnp.float32)
               + jnp.dot(h_ref[...], wr_ref[...],
                         preferred_element_type=jnp.float32)
               + bl_ref[...])
        o_ref[...] = jnp.where(out > 0, out, a_ref[...] * out)

    return pl.pallas_call(
        body,
        grid=(grid,),
        in_specs=[
            pl.BlockSpec((1, 1, blk, hd), lambda i: (0, 0, i, 0)),
            pl.BlockSpec((1, 1, blk, hd), lambda i: (0, 1, i, 0)),
            pl.BlockSpec((1, 1, blk, hd), lambda i: (1, 0, i, 0)),
            pl.BlockSpec((1, 1, blk, hd), lambda i: (1, 1, i, 0)),
            pl.BlockSpec((1, blk, CW), lambda i: (0, i, 0)),
            pl.BlockSpec((1, blk, CW), lambda i: (1, i, 0)),
            pl.BlockSpec((blk, d), lambda i: (i, 0)),
            pl.BlockSpec((d, d), lambda i: (0, 0)),
            pl.BlockSpec((1, d), lambda i: (0, 0)),
            pl.BlockSpec((d, d), lambda i: (0, 0)),
            pl.BlockSpec((1, d), lambda i: (0, 0)),
        ],
        out_specs=pl.BlockSpec((blk, d), lambda i: (i, 0)),
        out_shape=jax.ShapeDtypeStruct((n_nodes, d), jnp.float32),
    )


def kernel(x, edge_index, Wl0, bl0, Wr0, a0, Wl1, bl1, Wr1, a1,
           Wl2, bl2, Wr2, a2):
    n, d = x.shape
    e = edge_index.shape[1]

    tt = max(4, -(-e // (NS * CH)))
    t0 = max(1, round(tt * 0.57))
    t1 = tt - t0
    epad = NS * tt * CH + max(t0, t1) * CH
    zr = ZB * (-(-(n + 1) // (NS * ZB)))
    acc_rows = NS * zr

    ei = edge_index.astype(jnp.int32)
    pad = epad - e
    src = jnp.concatenate([ei[0], jnp.zeros((pad,), jnp.int32)])
    dst = jnp.concatenate([ei[1], jnp.full((pad,), n, jnp.int32)])

    sc_cnt = _make_sc_cnt(t0, t1, acc_rows, zr)
    sc_agg = _make_sc_agg(n, d, t0, t1, acc_rows, zr)
    tc = _make_tc_combine(n, d, 1000)

    hd = d // 2
    C = sc_cnt(dst)
    S = sc_agg(x[:, :hd], x[:, hd:], src, dst)
    h = tc(S, S, S, S, C, C, x,
           Wl0.T, bl0.reshape(1, d), Wr0.T, a0.reshape(1, d))
    S = sc_agg(h[:, :hd], h[:, hd:], src, dst)
    h = tc(S, S, S, S, C, C, h,
           Wl1.T, bl1.reshape(1, d), Wr1.T, a1.reshape(1, d))
    S = sc_agg(h[:, :hd], h[:, hd:], src, dst)
    h = tc(S, S, S, S, C, C, h,
           Wl2.T, bl2.reshape(1, d), Wr2.T, a2.reshape(1, d))
    return h
